# baseline (speedup 1.0000x reference)
import functools

import jax
import jax.numpy as jnp
from jax import lax
from jax.experimental import pallas as pl
from jax.experimental.pallas import tpu as pltpu
from jax.experimental.pallas import tpu_sc as plsc

EPS = 1e-7
B, N, K = 64, 2048, 16
F = 8
L = 16
NC, NS = 2, 16
NW = NC * NS
BPW = B // NW
NBLK = N // L


def _make_sc_kernel():
  mesh = plsc.VectorSubcoreMesh(
      core_axis_name="c", subcore_axis_name="s", num_cores=NC,
      num_subcores=NS)

  @functools.partial(
      pl.kernel,
      mesh=mesh,
      compiler_params=pltpu.CompilerParams(needs_layout_passes=False),
      out_type=jax.ShapeDtypeStruct((B, N * K), jnp.float32),
      scratch_types=[
          pltpu.VMEM((N * K,), jnp.float32),
          pltpu.VMEM((N * F,), jnp.float32),
          pltpu.VMEM((N * K,), jnp.float32),
          pltpu.VMEM((2 * K * L,), jnp.float32),
      ],
  )
  def sc_kernel(fracs_hbm, feats_hbm, out_hbm, fracs_v, feat_v, out_v,
                accT_v):
    wid = lax.axis_index("s") * NC + lax.axis_index("c")
    iota = lax.iota(jnp.int32, L)
    zero = jnp.zeros((L,), jnp.float32)

    for bi in range(BPW):
      b = wid * BPW + bi
      pltpu.sync_copy(fracs_hbm.at[b], fracs_v)
      pltpu.sync_copy(feats_hbm.at[b], feat_v)

      def blk(i, carry):
        accs0, accs1 = carry
        base = i * L
        fidx = (base + iota) * F
        e16 = plsc.load_gather(feat_v, [fidx])
        eta16 = plsc.load_gather(feat_v, [fidx + 1])
        ee16 = e16 * eta16
        cidx = (base + iota) * K
        n0 = []
        n1 = []
        for k in range(K):
          col = plsc.load_gather(fracs_v, [cidx + k])
          n0.append(accs0[k] + e16 * col)
          n1.append(accs1[k] + ee16 * col)
        return tuple(n0), tuple(n1)

      accs0, accs1 = lax.fori_loop(
          0, NBLK, blk, (tuple([zero] * K), tuple([zero] * K)))

      for k in range(K):
        accT_v[pl.ds(k * L, L)] = accs0[k]
        accT_v[pl.ds((K + k) * L, L)] = accs1[k]
      s0 = zero
      s1 = zero
      rowbase = iota * L
      for l in range(L):
        s0 = s0 + plsc.load_gather(accT_v, [rowbase + l])
        s1 = s1 + plsc.load_gather(accT_v, [rowbase + (K * L) + l])

      w = s1 / (s0 + EPS)
      w = jnp.where(jnp.abs(w) > 0.1, w, jnp.float32(500.0))

      accT_v[pl.ds(0, L)] = w
      rank = jnp.zeros((L,), jnp.int32)
      for j in range(K):
        bj = plsc.load_gather(accT_v, [jnp.full((L,), j, jnp.int32)])
        before = (bj < w) | ((bj == w) & (iota > j))
        rank = rank + jnp.where(before, 1, 0)

      def blk2(i, c):
        for t in range(L):
          off = (i * L + t) * K
          row = fracs_v[pl.ds(off, L)]
          plsc.store_scatter(out_v, [rank + off], row)
        return c

      lax.fori_loop(0, NBLK, blk2, 0)
      pltpu.sync_copy(out_v, out_hbm.at[b])

  return sc_kernel


_sc_kernel = _make_sc_kernel()


@jax.jit
def kernel(predicted_fracs, features):
  fracs2 = predicted_fracs.reshape(B, N * K)
  feats2 = features.reshape(B, N * F)
  out = _sc_kernel(fracs2, feats2)
  return out.reshape(B, N, K)


# speedup vs baseline: 1.1095x; 1.1095x over previous
import functools

import jax
import jax.numpy as jnp
from jax import lax
from jax.experimental import pallas as pl
from jax.experimental.pallas import tpu as pltpu
from jax.experimental.pallas import tpu_sc as plsc

EPS = 1e-7
B, N, K = 64, 2048, 16
F = 8
L = 16
NC, NS = 2, 16
NW = NC * NS
BPW = B // NW
NBLK = N // L

_DNUMS = lax.GatherDimensionNumbers(
    offset_dims=(), collapsed_slice_dims=(0,), start_index_map=(0,))


def _bcast(vec, lane):
  idx = jnp.full((L, 1), lane, dtype=jnp.int32)
  return lax.gather(vec, idx, _DNUMS, slice_sizes=(1,),
                    mode=lax.GatherScatterMode.PROMISE_IN_BOUNDS)


def _make_sc_kernel():
  mesh = plsc.VectorSubcoreMesh(
      core_axis_name="c", subcore_axis_name="s", num_cores=NC,
      num_subcores=NS)

  @functools.partial(
      pl.kernel,
      mesh=mesh,
      compiler_params=pltpu.CompilerParams(needs_layout_passes=False),
      out_type=jax.ShapeDtypeStruct((B, N * K), jnp.float32),
      scratch_types=[
          pltpu.VMEM((N * K,), jnp.float32),
          pltpu.VMEM((N * K,), jnp.float32),
          pltpu.VMEM((N * F,), jnp.float32),
          pltpu.VMEM((N * F,), jnp.float32),
          pltpu.SemaphoreType.DMA,
          pltpu.SemaphoreType.DMA,
          pltpu.SemaphoreType.DMA,
      ],
  )
  def sc_kernel(fracs_hbm, feats_hbm, out_hbm, fracs_v0, fracs_v1,
                feat_v0, feat_v1, sem_in0, sem_in1, sem_out):
    wid = lax.axis_index("s") * NC + lax.axis_index("c")
    iota = lax.iota(jnp.int32, L)
    zero = jnp.zeros((L,), jnp.float32)

    fracs_bufs = (fracs_v0, fracs_v1)
    feat_bufs = (feat_v0, feat_v1)
    in_sems = (sem_in0, sem_in1)

    b0 = wid * BPW
    in_copies = []
    for bi in range(BPW):
      in_copies.append(
          (pltpu.async_copy(fracs_hbm.at[b0 + bi], fracs_bufs[bi],
                            in_sems[bi]),
           pltpu.async_copy(feats_hbm.at[b0 + bi], feat_bufs[bi],
                            in_sems[bi])))

    out_copies = []
    for bi in range(BPW):
      fracs_v = fracs_bufs[bi]
      feat_v = feat_bufs[bi]
      for c in in_copies[bi]:
        c.wait()

      def blk(i, carry):
        acc0, acc1 = carry
        base = i * L
        fidx = (base + iota) * F
        e16 = plsc.load_gather(feat_v, [fidx])
        eta16 = plsc.load_gather(feat_v, [fidx + 1])
        ee16 = e16 * eta16
        rbase = base * K
        for t in range(L):
          row = fracs_v[pl.ds(rbase + t * K, L)]
          acc0 = acc0 + _bcast(e16, t) * row
          acc1 = acc1 + _bcast(ee16, t) * row
        return acc0, acc1

      s0, s1 = lax.fori_loop(0, NBLK, blk, (zero, zero))

      w = s1 / (s0 + EPS)
      w = jnp.where(jnp.abs(w) > 0.1, w, jnp.float32(500.0))

      rank = jnp.zeros((L,), jnp.int32)
      for j in range(K):
        bj = _bcast(w, j)
        before = (bj < w) | ((bj == w) & (iota > j))
        rank = rank + jnp.where(before, 1, 0)

      def blk2(i, c):
        for t in range(L):
          off = i * (L * K) + t * K
          row = fracs_v[pl.ds(off, L)]
          plsc.store_scatter(fracs_v, [rank + off], row)
        return c

      lax.fori_loop(0, NBLK, blk2, 0)
      out_copies.append(
          pltpu.async_copy(fracs_v, out_hbm.at[b0 + bi], sem_out))

    for c in out_copies:
      c.wait()

  return sc_kernel


_sc_kernel = _make_sc_kernel()


@jax.jit
def kernel(predicted_fracs, features):
  fracs2 = predicted_fracs.reshape(B, N * K)
  feats2 = features.reshape(B, N * F)
  out = _sc_kernel(fracs2, feats2)
  return out.reshape(B, N, K)


# speedup vs baseline: 2.3161x; 2.0876x over previous
import functools

import jax
import jax.numpy as jnp
from jax import lax
from jax.experimental import pallas as pl
from jax.experimental.pallas import tpu as pltpu
from jax.experimental.pallas import tpu_sc as plsc

EPS = 1e-7
B, N, K = 64, 2048, 16
F = 8
L = 16
NC, NS = 2, 16
NW = NC * NS
BPW = B // NW
NBLK = N // L
KG = 4
NSWEEP = K // KG

_DNUMS = lax.GatherDimensionNumbers(
    offset_dims=(), collapsed_slice_dims=(0,), start_index_map=(0,))


def _bcast(vec, lane):
  idx = jnp.full((L, 1), lane, dtype=jnp.int32)
  return lax.gather(vec, idx, _DNUMS, slice_sizes=(1,),
                    mode=lax.GatherScatterMode.PROMISE_IN_BOUNDS)


def _make_sc_kernel():
  mesh = plsc.VectorSubcoreMesh(
      core_axis_name="c", subcore_axis_name="s", num_cores=NC,
      num_subcores=NS)

  @functools.partial(
      pl.kernel,
      mesh=mesh,
      compiler_params=pltpu.CompilerParams(needs_layout_passes=False),
      out_type=jax.ShapeDtypeStruct((B * K, N), jnp.float32),
      scratch_types=[
          pltpu.VMEM((K, N), jnp.float32),
          pltpu.VMEM((K, N), jnp.float32),
          pltpu.VMEM((F, N), jnp.float32),
          pltpu.VMEM((K, N), jnp.float32),
          pltpu.VMEM((2 * K * L + L,), jnp.float32),
          pltpu.SemaphoreType.DMA,
          pltpu.SemaphoreType.DMA,
          pltpu.SemaphoreType.DMA,
          pltpu.SemaphoreType.DMA,
      ],
  )
  def sc_kernel(fracs_hbm, feats_hbm, out_hbm, fracs_v0, fracs_v1,
                feat_v, out_v, red_v, sem_f0, sem_f1, sem_e, sem_o):
    wid = lax.axis_index("s") * NC + lax.axis_index("c")
    iota = lax.iota(jnp.int32, L)
    zero = jnp.zeros((L,), jnp.float32)

    fracs_bufs = (fracs_v0, fracs_v1)
    fsems = (sem_f0, sem_f1)
    b0 = wid * BPW

    in_copies = [
        pltpu.async_copy(fracs_hbm.at[pl.ds((b0 + bi) * K, K)],
                         fracs_bufs[bi], fsems[bi])
        for bi in range(BPW)
    ]
    feat_copy = pltpu.async_copy(feats_hbm.at[pl.ds(b0 * F, F)], feat_v,
                                 sem_e)

    out_copy = None
    for bi in range(BPW):
      fracs_v = fracs_bufs[bi]
      in_copies[bi].wait()
      feat_copy.wait()

      sums0 = []
      sums1 = []
      for s in range(NSWEEP):
        def swp(i, carry, s=s):
          a0 = list(carry[0])
          a1 = list(carry[1])
          n0 = i * L
          e16 = feat_v[0, pl.ds(n0, L)]
          eta16 = feat_v[1, pl.ds(n0, L)]
          ee16 = e16 * eta16
          for g in range(KG):
            x = fracs_v[s * KG + g, pl.ds(n0, L)]
            a0[g] = a0[g] + e16 * x
            a1[g] = a1[g] + ee16 * x
          return tuple(a0), tuple(a1)

        a0, a1 = lax.fori_loop(
            0, NBLK, swp, (tuple([zero] * KG), tuple([zero] * KG)))
        sums0.extend(a0)
        sums1.extend(a1)

      for k in range(K):
        red_v[pl.ds(k * L, L)] = sums0[k]
        red_v[pl.ds((K + k) * L, L)] = sums1[k]
      s0 = zero
      s1 = zero
      rowbase = iota * L
      for l in range(L):
        s0 = s0 + plsc.load_gather(red_v, [rowbase + l])
        s1 = s1 + plsc.load_gather(red_v, [rowbase + (K * L) + l])

      if bi + 1 < BPW:
        feat_copy = pltpu.async_copy(
            feats_hbm.at[pl.ds((b0 + bi + 1) * F, F)], feat_v, sem_e)

      w = s1 / (s0 + EPS)
      w = jnp.where(jnp.abs(w) > 0.1, w, jnp.float32(500.0))

      rank = jnp.zeros((L,), jnp.int32)
      for j in range(K):
        bj = _bcast(w, j)
        before = (bj < w) | ((bj == w) & (iota > j))
        rank = rank + jnp.where(before, 1, 0)
      plsc.store_scatter(red_v, [rank + 2 * K * L],
                         jnp.asarray(iota, jnp.float32))
      perm = red_v[pl.ds(2 * K * L, L)]

      if out_copy is not None:
        out_copy.wait()

      def blk2(i, c, perm=perm):
        n0 = i * L
        cols = n0 + iota
        for r in range(K):
          src_row = jnp.asarray(_bcast(perm, r), jnp.int32)
          row = plsc.load_gather(fracs_v, [src_row, cols])
          out_v[r, pl.ds(n0, L)] = row
        return c

      lax.fori_loop(0, NBLK, blk2, 0)
      out_copy = pltpu.async_copy(out_v, out_hbm.at[pl.ds((b0 + bi) * K, K)],
                                  sem_o)

    out_copy.wait()

  return sc_kernel


_sc_kernel = _make_sc_kernel()


@jax.jit
def kernel(predicted_fracs, features):
  fracs_t = predicted_fracs.transpose(0, 2, 1).reshape(B * K, N)
  feats_t = features.transpose(0, 2, 1).reshape(B * F, N)
  out_t = _sc_kernel(fracs_t, feats_t)
  return out_t.reshape(B, K, N).transpose(0, 2, 1)


# speedup vs baseline: 3.1551x; 1.3622x over previous
import functools

import jax
import jax.numpy as jnp
from jax import lax
from jax.experimental import pallas as pl
from jax.experimental.pallas import tpu as pltpu
from jax.experimental.pallas import tpu_sc as plsc

EPS = 1e-7
B, N, K = 64, 2048, 16
F = 8
L = 16
NC, NS = 2, 16
NW = NC * NS
BPW = B // NW
NBLK = N // L
KG = 4
NSWEEP = K // KG

_DNUMS = lax.GatherDimensionNumbers(
    offset_dims=(), collapsed_slice_dims=(0,), start_index_map=(0,))


def _bcast(vec, lane):
  idx = jnp.full((L, 1), lane, dtype=jnp.int32)
  return lax.gather(vec, idx, _DNUMS, slice_sizes=(1,),
                    mode=lax.GatherScatterMode.PROMISE_IN_BOUNDS)


def _make_sc_kernel():
  mesh = plsc.VectorSubcoreMesh(
      core_axis_name="c", subcore_axis_name="s", num_cores=NC,
      num_subcores=NS)

  @functools.partial(
      pl.kernel,
      mesh=mesh,
      compiler_params=pltpu.CompilerParams(needs_layout_passes=False),
      out_type=jax.ShapeDtypeStruct((B * K, N), jnp.float32),
      scratch_types=[
          pltpu.VMEM((K, N), jnp.float32),
          pltpu.VMEM((K, N), jnp.float32),
          pltpu.VMEM((F, N), jnp.float32),
          pltpu.VMEM((K, N), jnp.float32),
          pltpu.VMEM((2 * K * L + L,), jnp.float32),
          pltpu.SemaphoreType.DMA,
          pltpu.SemaphoreType.DMA,
          pltpu.SemaphoreType.DMA,
          pltpu.SemaphoreType.DMA,
      ],
  )
  def sc_kernel(fracs_hbm, feats_hbm, out_hbm, fracs_v0, fracs_v1,
                feat_v, out_v, red_v, sem_f0, sem_f1, sem_e, sem_o):
    wid = lax.axis_index("s") * NC + lax.axis_index("c")
    iota = lax.iota(jnp.int32, L)
    zero = jnp.zeros((L,), jnp.float32)

    fracs_bufs = (fracs_v0, fracs_v1)
    fsems = (sem_f0, sem_f1)
    b0 = wid * BPW

    in_copies = [
        pltpu.async_copy(fracs_hbm.at[pl.ds((b0 + bi) * K, K)],
                         fracs_bufs[bi], fsems[bi])
        for bi in range(BPW)
    ]
    feat_copy = pltpu.async_copy(feats_hbm.at[pl.ds(b0 * F, F)], feat_v,
                                 sem_e)

    out_copy = None
    for bi in range(BPW):
      fracs_v = fracs_bufs[bi]
      in_copies[bi].wait()
      feat_copy.wait()

      sums0 = []
      sums1 = []
      for s in range(NSWEEP):
        def swp(i, carry, s=s):
          a0 = list(carry[0])
          a1 = list(carry[1])
          n0 = i * L
          e16 = feat_v[0, pl.ds(n0, L)]
          eta16 = feat_v[1, pl.ds(n0, L)]
          ee16 = e16 * eta16
          for g in range(KG):
            x = fracs_v[s * KG + g, pl.ds(n0, L)]
            a0[g] = a0[g] + e16 * x
            a1[g] = a1[g] + ee16 * x
          return tuple(a0), tuple(a1)

        a0, a1 = lax.fori_loop(
            0, NBLK, swp, (tuple([zero] * KG), tuple([zero] * KG)))
        sums0.extend(a0)
        sums1.extend(a1)

      for k in range(K):
        red_v[pl.ds(k * L, L)] = sums0[k]
        red_v[pl.ds((K + k) * L, L)] = sums1[k]
      s0 = zero
      s1 = zero
      rowbase = iota * L
      for l in range(L):
        s0 = s0 + plsc.load_gather(red_v, [rowbase + l])
        s1 = s1 + plsc.load_gather(red_v, [rowbase + (K * L) + l])

      if bi + 1 < BPW:
        feat_copy = pltpu.async_copy(
            feats_hbm.at[pl.ds((b0 + bi + 1) * F, F)], feat_v, sem_e)

      w = s1 / (s0 + EPS)
      w = jnp.where(jnp.abs(w) > 0.1, w, jnp.float32(500.0))

      rank = jnp.zeros((L,), jnp.int32)
      for j in range(K):
        bj = _bcast(w, j)
        before = (bj < w) | ((bj == w) & (iota > j))
        rank = rank + jnp.where(before, 1, 0)
      branks = [_bcast(rank, k) for k in range(K)]

      if out_copy is not None:
        out_copy.wait()

      def blk2(i, c, branks=branks):
        n0 = i * L
        cols = n0 + iota
        rows = [fracs_v[k, pl.ds(n0, L)] for k in range(K)]
        for k in range(K):
          plsc.store_scatter(out_v, [branks[k], cols], rows[k])
        return c

      lax.fori_loop(0, NBLK, blk2, 0)
      out_copy = pltpu.async_copy(out_v, out_hbm.at[pl.ds((b0 + bi) * K, K)],
                                  sem_o)

    out_copy.wait()

  return sc_kernel


_sc_kernel = _make_sc_kernel()


@jax.jit
def kernel(predicted_fracs, features):
  fracs_t = predicted_fracs.transpose(0, 2, 1).reshape(B * K, N)
  feats_t = features.transpose(0, 2, 1).reshape(B * F, N)
  out_t = _sc_kernel(fracs_t, feats_t)
  return out_t.reshape(B, K, N).transpose(0, 2, 1)
